# baseline (speedup 1.0000x reference)
import functools

import jax
import jax.numpy as jnp
from jax import lax
from jax.experimental import pallas as pl
from jax.experimental.pallas import tpu as pltpu
from jax.experimental.pallas import tpu_sc as plsc

N = 10000
E = 320000
D_IN = 128
H = 32
DT = 2

NPAD = 10112
CHUNK = 512
ROWS = 640
E_PAD = ROWS * CHUNK
NC = 2
NS = 16
NW = NC * NS
RPW = ROWS // NW
RPS = NPAD // NS
EQ = E_PAD // 4
BEQ = 512
NBLK = EQ // BEQ
NP4 = NPAD // 4
NV4 = N // 4
EPS = 1e-5

_Z16 = functools.partial(jnp.zeros, (16,), jnp.float32)

_SC_PARAMS = pltpu.CompilerParams(use_tc_tiling_on_sc=False)


def _node_body(x_ref, Wf1_ref, bf1, gf1, bef1, Wf2_ref, bf2, gf2, bef2,
               We1_ref, be1, P_ref, Q_ref, sum_ref):
    x = x_ref[...]
    a = jnp.dot(x, Wf1_ref[...], preferred_element_type=jnp.float32) + bf1[...]
    m = jnp.mean(a, axis=0, keepdims=True)
    v = jnp.mean((a - m) ** 2, axis=0, keepdims=True)
    h = jnp.maximum((a - m) * lax.rsqrt(v + EPS) * gf1[...] + bef1[...], 0.0)
    a2 = jnp.dot(h, Wf2_ref[...], preferred_element_type=jnp.float32) + bf2[...]
    m2 = jnp.mean(a2, axis=0, keepdims=True)
    v2 = jnp.mean((a2 - m2) ** 2, axis=0, keepdims=True)
    h0 = jnp.maximum((a2 - m2) * lax.rsqrt(v2 + EPS) * gf2[...] + bef2[...], 0.0)
    A = We1_ref[0:H, :]
    B = We1_ref[H:2 * H, :]
    P_ref[0:N, :] = jnp.dot(h0, A - B, preferred_element_type=jnp.float32) + be1[...]
    P_ref[N:NPAD, :] = jnp.zeros((NPAD - N, H), jnp.float32)
    Q_ref[0:N, :] = jnp.dot(h0, B, preferred_element_type=jnp.float32)
    Q_ref[N:NPAD, :] = jnp.zeros((NPAD - N, H), jnp.float32)
    sum_ref[...] = jnp.sum(h0, axis=0, keepdims=True)


def _node_stage(x, Wf1, bf1, gf1, bef1, Wf2, bf2, gf2, bef2, We1, be1):
    return pl.pallas_call(
        _node_body,
        out_shape=(
            jax.ShapeDtypeStruct((NPAD, H), jnp.float32),
            jax.ShapeDtypeStruct((NPAD, H), jnp.float32),
            jax.ShapeDtypeStruct((1, H), jnp.float32),
        ),
    )(x, Wf1, bf1, gf1, bef1, Wf2, bf2, gf2, bef2, We1, be1)


@functools.cache
def _sc_mesh():
    return plsc.VectorSubcoreMesh(core_axis_name="c", subcore_axis_name="s",
                                  num_cores=NC, num_subcores=NS)


@functools.cache
def _sc_gather_fn():
    @functools.partial(
        pl.kernel,
        out_type=(
            jax.ShapeDtypeStruct((E_PAD, H), jnp.float32),
            jax.ShapeDtypeStruct((NC, NPAD, H), jnp.float32),
        ),
        mesh=_sc_mesh(),
        scratch_types=[
            pltpu.VMEM((RPW, CHUNK), jnp.int32),
            pltpu.VMEM((RPW, CHUNK), jnp.int32),
            pltpu.VMEM((CHUNK, H), jnp.float32),
            pltpu.VMEM((CHUNK, H), jnp.float32),
            pltpu.VMEM((RPS, H), jnp.float32),
            pltpu.VMEM_SHARED((NPAD, H), jnp.float32),
            pltpu.SemaphoreType.DMA,
        ],
        compiler_params=_SC_PARAMS,
    )
    def _sc_gather(P_hbm, Q_hbm, dst_hbm, src_hbm,
                   l1_hbm, cnt_hbm,
                   dstv, srcv, rows, ones_v, zbuf, cnt_sh, semP):
        c = lax.axis_index("c")
        s = lax.axis_index("s")
        w = s * NC + c
        base_row = w * RPW

        pltpu.sync_copy(dst_hbm.at[pl.ds(base_row, RPW)], dstv)
        pltpu.sync_copy(src_hbm.at[pl.ds(base_row, RPW)], srcv)

        def _fill(i, _):
            ones_v[i, 0:16] = jnp.ones((16,), jnp.float32)
            ones_v[i, 16:32] = jnp.ones((16,), jnp.float32)
            return 0
        lax.fori_loop(0, CHUNK, _fill, 0)

        def _zero(i, _):
            zbuf[i, 0:16] = _Z16()
            zbuf[i, 16:32] = _Z16()
            return 0
        lax.fori_loop(0, RPS, _zero, 0)
        pltpu.sync_copy(zbuf, cnt_sh.at[pl.ds(s * RPS, RPS)])
        plsc.subcore_barrier()

        def _chunk(j, _):
            cpP = pltpu.async_copy(P_hbm.at[dstv.at[j]], rows, semP)
            cpP.wait()
            pltpu.sync_copy(Q_hbm.at[srcv.at[j]], rows, add=True)
            pltpu.sync_copy(ones_v, cnt_sh.at[dstv.at[j]], add=True)
            pltpu.sync_copy(rows,
                            l1_hbm.at[pl.ds((base_row + j) * CHUNK, CHUNK)])
            return 0

        lax.fori_loop(0, RPW, _chunk, 0)

        plsc.subcore_barrier()
        pltpu.sync_copy(cnt_sh.at[pl.ds(s * RPS, RPS)],
                        cnt_hbm.at[c, pl.ds(s * RPS, RPS)])

    return _sc_gather


def _stats_body(l1_ref, ge1, bee1, W4_ref, be2p, ge2, bee2,
                st_ref, s1sum, s1sq, ssum, ssq):
    pid = pl.program_id(0)
    blk = l1_ref[...]

    def fold(a):
        return (a[:, 0:H] + a[:, H:2 * H]
                + a[:, 2 * H:3 * H] + a[:, 3 * H:4 * H])

    @pl.when(pid == 0)
    def _():
        s1sum[...] = jnp.zeros_like(s1sum)
        s1sq[...] = jnp.zeros_like(s1sq)
        ssum[...] = jnp.zeros_like(ssum)
        ssq[...] = jnp.zeros_like(ssq)

    @pl.when(pid < NBLK)
    def _():
        s1sum[...] += jnp.sum(blk, axis=0, keepdims=True)
        s1sq[...] += jnp.sum(blk * blk, axis=0, keepdims=True)

    @pl.when(pid == NBLK - 1)
    def _():
        m1 = fold(s1sum[...]) / E
        v1 = fold(s1sq[...]) / E - m1 * m1
        s1 = ge1[...] * lax.rsqrt(v1 + EPS)
        t1 = bee1[...] - m1 * s1
        st_ref[0:1, :] = jnp.concatenate([s1, s1, s1, s1], axis=1)
        st_ref[1:2, :] = jnp.concatenate([t1, t1, t1, t1], axis=1)

    @pl.when(pid >= NBLK)
    def _():
        h = jnp.maximum(blk * st_ref[0:1, :] + st_ref[1:2, :], 0.0)
        l2 = jnp.dot(h, W4_ref[...],
                     preferred_element_type=jnp.float32) + be2p[...]
        rows = lax.broadcasted_iota(jnp.int32, (BEQ, 128), 0)
        grp = lax.broadcasted_iota(jnp.int32, (BEQ, 128), 1) // H
        eid = 4 * ((pid - NBLK) * BEQ + rows) + grp
        l2m = jnp.where(eid < E, l2, 0.0)
        ssum[...] += jnp.sum(l2m, axis=0, keepdims=True)
        ssq[...] += jnp.sum(l2m * l2m, axis=0, keepdims=True)

    @pl.when(pid == 2 * NBLK - 1)
    def _():
        m2 = fold(ssum[...]) / E
        v2 = fold(ssq[...]) / E - m2 * m2
        s2 = ge2[...] * lax.rsqrt(v2 + EPS)
        t2 = bee2[...] - m2 * s2
        st_ref[2:3, :] = jnp.concatenate([s2, s2, s2, s2], axis=1)
        st_ref[3:4, :] = jnp.concatenate([t2, t2, t2, t2], axis=1)


def _stats_stage(l1p, ge1, bee1, W4, be2p, ge2, bee2):
    return pl.pallas_call(
        _stats_body,
        grid=(2 * NBLK,),
        in_specs=[
            pl.BlockSpec((BEQ, 128), lambda i: (i % NBLK, 0)),
            pl.BlockSpec((1, H), lambda i: (0, 0)),
            pl.BlockSpec((1, H), lambda i: (0, 0)),
            pl.BlockSpec((128, 128), lambda i: (0, 0)),
            pl.BlockSpec((1, 128), lambda i: (0, 0)),
            pl.BlockSpec((1, H), lambda i: (0, 0)),
            pl.BlockSpec((1, H), lambda i: (0, 0)),
        ],
        out_specs=[
            pl.BlockSpec((4, 128), lambda i: (0, 0)),
        ],
        out_shape=(
            jax.ShapeDtypeStruct((4, 128), jnp.float32),
        ),
        scratch_shapes=[
            pltpu.VMEM((1, 128), jnp.float32),
            pltpu.VMEM((1, 128), jnp.float32),
            pltpu.VMEM((1, 128), jnp.float32),
            pltpu.VMEM((1, 128), jnp.float32),
        ],
    )(l1p, ge1, bee1, W4, be2p, ge2, bee2)


G = 4
EQG = EQ // G
NBLKG = NBLK // G
ROWSG = ROWS // G
RPWG = ROWSG // NW


def _mh_body(l1_ref, st_ref, W4_ref, be2p, mh_ref):
    blk = l1_ref[...]
    h = jnp.maximum(blk * st_ref[0:1, :] + st_ref[1:2, :], 0.0)
    l2 = jnp.dot(h, W4_ref[...],
                 preferred_element_type=jnp.float32) + be2p[...]
    mh_ref[...] = jnp.maximum(l2 * st_ref[2:3, :] + st_ref[3:4, :], 0.0)


def _mh_stage(l1g, st, W4, be2p):
    return pl.pallas_call(
        _mh_body,
        grid=(NBLKG,),
        in_specs=[
            pl.BlockSpec((BEQ, 128), lambda i: (i, 0)),
            pl.BlockSpec((4, 128), lambda i: (0, 0)),
            pl.BlockSpec((128, 128), lambda i: (0, 0)),
            pl.BlockSpec((1, 128), lambda i: (0, 0)),
        ],
        out_specs=[
            pl.BlockSpec((BEQ, 128), lambda i: (i, 0)),
        ],
        out_shape=(
            jax.ShapeDtypeStruct((EQG, 128), jnp.float32),
        ),
    )(l1g, st, W4, be2p)


@functools.cache
def _sc_scatter_fn():
    @functools.partial(
        pl.kernel,
        out_type=jax.ShapeDtypeStruct((NC, NPAD, H), jnp.float32),
        mesh=_sc_mesh(),
        scratch_types=[
            pltpu.VMEM((RPWG, CHUNK), jnp.int32),
            pltpu.VMEM((CHUNK, H), jnp.float32),
            pltpu.VMEM((RPS, H), jnp.float32),
            pltpu.VMEM_SHARED((NPAD, H), jnp.float32),
        ],
        compiler_params=_SC_PARAMS,
    )
    def _sc_scatter(mh_hbm, dst_hbm, S_hbm,
                    dstv, rows, zbuf, S_sh):
        c = lax.axis_index("c")
        s = lax.axis_index("s")
        w = s * NC + c
        base_row = w * RPWG

        pltpu.sync_copy(dst_hbm.at[pl.ds(base_row, RPWG)], dstv)

        def _zero(i, _):
            zbuf[i, 0:16] = _Z16()
            zbuf[i, 16:32] = _Z16()
            return 0
        lax.fori_loop(0, RPS, _zero, 0)
        pltpu.sync_copy(zbuf, S_sh.at[pl.ds(s * RPS, RPS)])
        plsc.subcore_barrier()

        def _chunk(j, _):
            pltpu.sync_copy(mh_hbm.at[pl.ds((base_row + j) * CHUNK, CHUNK)],
                            rows)
            pltpu.sync_copy(rows, S_sh.at[dstv.at[j]], add=True)
            return 0

        lax.fori_loop(0, RPWG, _chunk, 0)
        plsc.subcore_barrier()
        pltpu.sync_copy(S_sh.at[pl.ds(s * RPS, RPS)],
                        S_hbm.at[c, pl.ds(s * RPS, RPS)])

    return _sc_scatter


def _final_body(S0_ref, S1_ref, S2_ref, S3_ref, cnt_ref, sumh0,
                Wl0_ref, bl0, Wl1_ref, bl1, out_ref):
    Sv = ((S0_ref[0:NV4, :] + S0_ref[NP4:NP4 + NV4, :])
          + (S1_ref[0:NV4, :] + S1_ref[NP4:NP4 + NV4, :])
          + (S2_ref[0:NV4, :] + S2_ref[NP4:NP4 + NV4, :])
          + (S3_ref[0:NV4, :] + S3_ref[NP4:NP4 + NV4, :]))
    cv = cnt_ref[0:NV4, :] + cnt_ref[NP4:NP4 + NV4, :]
    h1 = Sv / jnp.maximum(cv, 1.0)
    t = jnp.sum(h1, axis=0, keepdims=True)
    mh1 = (t[:, 0:H] + t[:, H:2 * H]
           + t[:, 2 * H:3 * H] + t[:, 3 * H:4 * H]) / N
    mh0 = sumh0[...] / N
    out_ref[...] = (jnp.dot(mh0, Wl0_ref[...], preferred_element_type=jnp.float32)
                    + bl0[...]
                    + jnp.dot(mh1, Wl1_ref[...], preferred_element_type=jnp.float32)
                    + bl1[...])


def _final_stage(Ss, cnt, sumh0, Wl0, bl0, Wl1, bl1):
    return pl.pallas_call(
        _final_body,
        out_shape=jax.ShapeDtypeStruct((1, DT), jnp.float32),
    )(*Ss, cnt, sumh0, Wl0, bl0, Wl1, bl1)


def kernel(x, Wf1, bf1, gf1, betaf1, Wf2, bf2, gf2, betaf2, Wl0, bl0,
           We1, be1, ge1, betae1, We2, be2, ge2, betae2, Wl1, bl1,
           edge_index):
    r = lambda p: p.reshape(1, -1)
    src = edge_index[0]
    dst = edge_index[1]
    pad = N + (jnp.arange(E_PAD - E, dtype=jnp.int32) % (NPAD - N))
    dst2 = jnp.concatenate([dst, pad]).reshape(ROWS, CHUNK)
    src2 = jnp.concatenate([src, pad]).reshape(ROWS, CHUNK)
    W4 = jnp.kron(jnp.eye(4, dtype=jnp.float32), We2)
    be2p = jnp.tile(be2.reshape(1, -1), (1, 4))

    P, Q, sumh0 = _node_stage(x, Wf1, r(bf1), r(gf1), r(betaf1),
                              Wf2, r(bf2), r(gf2), r(betaf2), We1, r(be1))
    l1, cnt = _sc_gather_fn()(P, Q, dst2, src2)
    l1p = l1.reshape(EQ, 128)
    (st,) = _stats_stage(l1p, r(ge1), r(betae1), W4, be2p, r(ge2), r(betae2))
    Ss = []
    for g in range(G):
        (mh_g,) = _mh_stage(l1p[g * EQG:(g + 1) * EQG], st, W4, be2p)
        S_g = _sc_scatter_fn()(mh_g.reshape(E_PAD // G, H),
                               dst2[g * ROWSG:(g + 1) * ROWSG])
        Ss.append(S_g.reshape(NC * NP4, 128))
    out = _final_stage(Ss, cnt.reshape(NC * NP4, 128),
                       sumh0, Wl0, r(bl0), Wl1, r(bl1))
    return out


# speedup vs baseline: 1.7013x; 1.7013x over previous
import functools

import jax
import jax.numpy as jnp
from jax import lax
from jax.experimental import pallas as pl
from jax.experimental.pallas import tpu as pltpu
from jax.experimental.pallas import tpu_sc as plsc

N = 10000
E = 320000
D_IN = 128
H = 32
DT = 2

NPAD = 10112
CHUNK = 512
ROWS = 640
E_PAD = ROWS * CHUNK
NC = 2
NS = 16
NW = NC * NS
RPW = ROWS // NW
RPS = NPAD // NS
EQ = E_PAD // 4
BEQ = 512
NBLK = EQ // BEQ
NP4 = NPAD // 4
NV4 = N // 4
EPS = 1e-5

_Z16 = functools.partial(jnp.zeros, (16,), jnp.float32)

_SC_PARAMS = pltpu.CompilerParams(use_tc_tiling_on_sc=False)


def _node_body(x_ref, Wf1_ref, bf1, gf1, bef1, Wf2_ref, bf2, gf2, bef2,
               We1_ref, be1, P_ref, Q_ref, sum_ref):
    x = x_ref[...]
    a = jnp.dot(x, Wf1_ref[...], preferred_element_type=jnp.float32) + bf1[...]
    m = jnp.mean(a, axis=0, keepdims=True)
    v = jnp.mean((a - m) ** 2, axis=0, keepdims=True)
    h = jnp.maximum((a - m) * lax.rsqrt(v + EPS) * gf1[...] + bef1[...], 0.0)
    a2 = jnp.dot(h, Wf2_ref[...], preferred_element_type=jnp.float32) + bf2[...]
    m2 = jnp.mean(a2, axis=0, keepdims=True)
    v2 = jnp.mean((a2 - m2) ** 2, axis=0, keepdims=True)
    h0 = jnp.maximum((a2 - m2) * lax.rsqrt(v2 + EPS) * gf2[...] + bef2[...], 0.0)
    A = We1_ref[0:H, :]
    B = We1_ref[H:2 * H, :]
    P_ref[0:N, :] = jnp.dot(h0, A - B, preferred_element_type=jnp.float32) + be1[...]
    P_ref[N:NPAD, :] = jnp.zeros((NPAD - N, H), jnp.float32)
    Q_ref[0:N, :] = jnp.dot(h0, B, preferred_element_type=jnp.float32)
    Q_ref[N:NPAD, :] = jnp.zeros((NPAD - N, H), jnp.float32)
    sum_ref[...] = jnp.sum(h0, axis=0, keepdims=True)


def _node_stage(x, Wf1, bf1, gf1, bef1, Wf2, bf2, gf2, bef2, We1, be1):
    return pl.pallas_call(
        _node_body,
        out_shape=(
            jax.ShapeDtypeStruct((NPAD, H), jnp.float32),
            jax.ShapeDtypeStruct((NPAD, H), jnp.float32),
            jax.ShapeDtypeStruct((1, H), jnp.float32),
        ),
    )(x, Wf1, bf1, gf1, bef1, Wf2, bf2, gf2, bef2, We1, be1)


@functools.cache
def _sc_mesh():
    return plsc.VectorSubcoreMesh(core_axis_name="c", subcore_axis_name="s",
                                  num_cores=NC, num_subcores=NS)


@functools.cache
def _sc_gather_fn():
    @functools.partial(
        pl.kernel,
        out_type=(
            jax.ShapeDtypeStruct((E_PAD, H), jnp.float32),
            jax.ShapeDtypeStruct((NC, NPAD, H), jnp.float32),
        ),
        mesh=_sc_mesh(),
        scratch_types=[
            pltpu.VMEM((RPW, CHUNK), jnp.int32),
            pltpu.VMEM((RPW, CHUNK), jnp.int32),
            pltpu.VMEM((CHUNK, H), jnp.float32),
            pltpu.VMEM((CHUNK, H), jnp.float32),
            pltpu.VMEM((RPS, H), jnp.float32),
            pltpu.VMEM_SHARED((NPAD, H), jnp.float32),
            pltpu.SemaphoreType.DMA,
        ],
        compiler_params=_SC_PARAMS,
    )
    def _sc_gather(P_hbm, Q_hbm, dst_hbm, src_hbm,
                   l1_hbm, cnt_hbm,
                   dstv, srcv, rows, ones_v, zbuf, cnt_sh, semP):
        c = lax.axis_index("c")
        s = lax.axis_index("s")
        w = s * NC + c
        base_row = w * RPW

        pltpu.sync_copy(dst_hbm.at[pl.ds(base_row, RPW)], dstv)
        pltpu.sync_copy(src_hbm.at[pl.ds(base_row, RPW)], srcv)

        def _fill(i, _):
            ones_v[i, 0:16] = jnp.ones((16,), jnp.float32)
            ones_v[i, 16:32] = jnp.ones((16,), jnp.float32)
            return 0
        lax.fori_loop(0, CHUNK, _fill, 0)

        def _zero(i, _):
            zbuf[i, 0:16] = _Z16()
            zbuf[i, 16:32] = _Z16()
            return 0
        lax.fori_loop(0, RPS, _zero, 0)
        pltpu.sync_copy(zbuf, cnt_sh.at[pl.ds(s * RPS, RPS)])
        plsc.subcore_barrier()

        def _chunk(j, _):
            cpP = pltpu.async_copy(P_hbm.at[dstv.at[j]], rows, semP)
            cpP.wait()
            pltpu.sync_copy(Q_hbm.at[srcv.at[j]], rows, add=True)
            pltpu.sync_copy(ones_v, cnt_sh.at[dstv.at[j]], add=True)
            pltpu.sync_copy(rows,
                            l1_hbm.at[pl.ds((base_row + j) * CHUNK, CHUNK)])
            return 0

        lax.fori_loop(0, RPW, _chunk, 0)

        plsc.subcore_barrier()
        pltpu.sync_copy(cnt_sh.at[pl.ds(s * RPS, RPS)],
                        cnt_hbm.at[c, pl.ds(s * RPS, RPS)])

    return _sc_gather


def _edge_body(l1_ref, ge1, bee1, W4_ref, be2p, ge2, bee2,
               mh_ref, buf, st, s1sum, s1sq, ssum, ssq):
    pid = pl.program_id(0)

    def fold(a):
        return (a[:, 0:H] + a[:, H:2 * H]
                + a[:, 2 * H:3 * H] + a[:, 3 * H:4 * H])

    @pl.when(pid < NBLK)
    def _():
        @pl.when(pid == 0)
        def _():
            s1sum[...] = jnp.zeros_like(s1sum)
            s1sq[...] = jnp.zeros_like(s1sq)
            ssum[...] = jnp.zeros_like(ssum)
            ssq[...] = jnp.zeros_like(ssq)

        blk = l1_ref[...]
        s1sum[...] += jnp.sum(blk, axis=0, keepdims=True)
        s1sq[...] += jnp.sum(blk * blk, axis=0, keepdims=True)
        buf[pl.ds(pid * BEQ, BEQ), :] = blk

        @pl.when(pid == NBLK - 1)
        def _():
            m1 = fold(s1sum[...]) / E
            v1 = fold(s1sq[...]) / E - m1 * m1
            s1 = ge1[...] * lax.rsqrt(v1 + EPS)
            t1 = bee1[...] - m1 * s1
            st[0:1, :] = jnp.concatenate([s1, s1, s1, s1], axis=1)
            st[1:2, :] = jnp.concatenate([t1, t1, t1, t1], axis=1)

    @pl.when((pid >= NBLK) & (pid < 2 * NBLK))
    def _():
        b = pid - NBLK
        x = buf[pl.ds(b * BEQ, BEQ), :]
        h = jnp.maximum(x * st[0:1, :] + st[1:2, :], 0.0)
        l2 = jnp.dot(h, W4_ref[...],
                     preferred_element_type=jnp.float32) + be2p[...]
        buf[pl.ds(b * BEQ, BEQ), :] = l2
        rows = lax.broadcasted_iota(jnp.int32, (BEQ, 128), 0)
        grp = lax.broadcasted_iota(jnp.int32, (BEQ, 128), 1) // H
        eid = 4 * (b * BEQ + rows) + grp
        l2m = jnp.where(eid < E, l2, 0.0)
        ssum[...] += jnp.sum(l2m, axis=0, keepdims=True)
        ssq[...] += jnp.sum(l2m * l2m, axis=0, keepdims=True)

        @pl.when(pid == 2 * NBLK - 1)
        def _():
            m2 = fold(ssum[...]) / E
            v2 = fold(ssq[...]) / E - m2 * m2
            s2 = ge2[...] * lax.rsqrt(v2 + EPS)
            t2 = bee2[...] - m2 * s2
            st[2:3, :] = jnp.concatenate([s2, s2, s2, s2], axis=1)
            st[3:4, :] = jnp.concatenate([t2, t2, t2, t2], axis=1)

    @pl.when(pid >= 2 * NBLK)
    def _():
        b = pid - 2 * NBLK
        l2 = buf[pl.ds(b * BEQ, BEQ), :]
        mh_ref[...] = jnp.maximum(l2 * st[2:3, :] + st[3:4, :], 0.0)


def _edge_stage(l1p, ge1, bee1, W4, be2p, ge2, bee2):
    return pl.pallas_call(
        _edge_body,
        grid=(3 * NBLK,),
        in_specs=[
            pl.BlockSpec((BEQ, 128),
                         lambda i: (jnp.where(i < NBLK, i, 0), 0)),
            pl.BlockSpec((1, H), lambda i: (0, 0)),
            pl.BlockSpec((1, H), lambda i: (0, 0)),
            pl.BlockSpec((128, 128), lambda i: (0, 0)),
            pl.BlockSpec((1, 128), lambda i: (0, 0)),
            pl.BlockSpec((1, H), lambda i: (0, 0)),
            pl.BlockSpec((1, H), lambda i: (0, 0)),
        ],
        out_specs=[
            pl.BlockSpec((BEQ, 128),
                         lambda i: (jnp.where(i < 2 * NBLK, 0, i - 2 * NBLK),
                                    0)),
        ],
        out_shape=(
            jax.ShapeDtypeStruct((EQ, 128), jnp.float32),
        ),
        scratch_shapes=[
            pltpu.VMEM((EQ, 128), jnp.float32),
            pltpu.VMEM((4, 128), jnp.float32),
            pltpu.VMEM((1, 128), jnp.float32),
            pltpu.VMEM((1, 128), jnp.float32),
            pltpu.VMEM((1, 128), jnp.float32),
            pltpu.VMEM((1, 128), jnp.float32),
        ],
    )(l1p, ge1, bee1, W4, be2p, ge2, bee2)


@functools.cache
def _sc_scatter_fn():
    @functools.partial(
        pl.kernel,
        out_type=jax.ShapeDtypeStruct((NC, NPAD, H), jnp.float32),
        mesh=_sc_mesh(),
        scratch_types=[
            pltpu.VMEM((RPW, CHUNK), jnp.int32),
            pltpu.VMEM((CHUNK, H), jnp.float32),
            pltpu.VMEM((RPS, H), jnp.float32),
            pltpu.VMEM_SHARED((NPAD, H), jnp.float32),
        ],
        compiler_params=_SC_PARAMS,
    )
    def _sc_scatter(mh_hbm, dst_hbm, S_hbm,
                    dstv, rows, zbuf, S_sh):
        c = lax.axis_index("c")
        s = lax.axis_index("s")
        w = s * NC + c
        base_row = w * RPW

        pltpu.sync_copy(dst_hbm.at[pl.ds(base_row, RPW)], dstv)

        def _zero(i, _):
            zbuf[i, 0:16] = _Z16()
            zbuf[i, 16:32] = _Z16()
            return 0
        lax.fori_loop(0, RPS, _zero, 0)
        pltpu.sync_copy(zbuf, S_sh.at[pl.ds(s * RPS, RPS)])
        plsc.subcore_barrier()

        def _chunk(j, _):
            pltpu.sync_copy(mh_hbm.at[pl.ds((base_row + j) * CHUNK, CHUNK)],
                            rows)
            pltpu.sync_copy(rows, S_sh.at[dstv.at[j]], add=True)
            return 0

        lax.fori_loop(0, RPW, _chunk, 0)
        plsc.subcore_barrier()
        pltpu.sync_copy(S_sh.at[pl.ds(s * RPS, RPS)],
                        S_hbm.at[c, pl.ds(s * RPS, RPS)])

    return _sc_scatter


def _final_body(S_ref, cnt_ref, sumh0,
                Wl0_ref, bl0, Wl1_ref, bl1, out_ref):
    Sv = S_ref[0:NV4, :] + S_ref[NP4:NP4 + NV4, :]
    cv = cnt_ref[0:NV4, :] + cnt_ref[NP4:NP4 + NV4, :]
    h1 = Sv / jnp.maximum(cv, 1.0)
    t = jnp.sum(h1, axis=0, keepdims=True)
    mh1 = (t[:, 0:H] + t[:, H:2 * H]
           + t[:, 2 * H:3 * H] + t[:, 3 * H:4 * H]) / N
    mh0 = sumh0[...] / N
    out_ref[...] = (jnp.dot(mh0, Wl0_ref[...], preferred_element_type=jnp.float32)
                    + bl0[...]
                    + jnp.dot(mh1, Wl1_ref[...], preferred_element_type=jnp.float32)
                    + bl1[...])


def _final_stage(S, cnt, sumh0, Wl0, bl0, Wl1, bl1):
    return pl.pallas_call(
        _final_body,
        out_shape=jax.ShapeDtypeStruct((1, DT), jnp.float32),
    )(S, cnt, sumh0, Wl0, bl0, Wl1, bl1)


def kernel(x, Wf1, bf1, gf1, betaf1, Wf2, bf2, gf2, betaf2, Wl0, bl0,
           We1, be1, ge1, betae1, We2, be2, ge2, betae2, Wl1, bl1,
           edge_index):
    r = lambda p: p.reshape(1, -1)
    src = edge_index[0]
    dst = edge_index[1]
    pad = N + (jnp.arange(E_PAD - E, dtype=jnp.int32) % (NPAD - N))
    dst2 = jnp.concatenate([dst, pad]).reshape(ROWS, CHUNK)
    src2 = jnp.concatenate([src, pad]).reshape(ROWS, CHUNK)
    W4 = jnp.kron(jnp.eye(4, dtype=jnp.float32), We2)
    be2p = jnp.tile(be2.reshape(1, -1), (1, 4))

    P, Q, sumh0 = _node_stage(x, Wf1, r(bf1), r(gf1), r(betaf1),
                              Wf2, r(bf2), r(gf2), r(betaf2), We1, r(be1))
    l1, cnt = _sc_gather_fn()(P, Q, dst2, src2)
    (mhp,) = _edge_stage(l1.reshape(EQ, 128), r(ge1), r(betae1),
                         W4, be2p, r(ge2), r(betae2))
    S = _sc_scatter_fn()(mhp.reshape(E_PAD, H), dst2)
    out = _final_stage(S.reshape(NC * NP4, 128), cnt.reshape(NC * NP4, 128),
                       sumh0, Wl0, r(bl0), Wl1, r(bl1))
    return out
